# Initial kernel scaffold; baseline (speedup 1.0000x reference)
import jax
import jax.numpy as jnp
from jax.experimental import pallas as pl


def kernel(x, mask_real, mask_imag, W1, b1, W2, b2):
    raise NotImplementedError("write your pallas kernel here")


# speedup vs baseline: 1.2810x; 1.2810x over previous
import numpy as np
import jax
import jax.numpy as jnp
from jax.experimental import pallas as pl
from jax.experimental.pallas import tpu as pltpu

_N = 2048
_F = 1025
_FP = 1032
_C = 128

_K_LO = 716 + (_FP - _F)

_IDX = np.float32(0.7) * np.float32(_F - 1)
_LW = np.float32(np.float32(717.0) - _IDX)
_HW = np.float32(_IDX - np.float32(716.0))


def _build_dft():
    f = np.arange(_FP, dtype=np.int64)[:, None]
    w = np.arange(_N, dtype=np.int64)[None, :]
    ang = (2.0 * np.pi / _N) * ((f * w) % _N).astype(np.float64)
    cos = np.cos(ang)
    sin = np.sin(ang)
    cos[_F:, :] = 0.0
    sin[_F:, :] = 0.0
    fwd = np.concatenate([cos, -sin], axis=0).astype(np.float32)
    wgt = np.full((_FP, 1), 2.0 / _N)
    wgt[0] = 1.0 / _N
    wgt[_N // 2] = 1.0 / _N
    wgt[_F:] = 0.0
    inv_c = (wgt * cos).T.astype(np.float32)
    inv_s = (-(wgt * sin)).T.astype(np.float32)
    return fwd, inv_c, inv_s


_FWD_NP, _INVC_NP, _INVS_NP = _build_dft()


def _body(x_ref, fwd_ref, invc_ref, invs_ref, mr_ref, mi_ref,
          w1_ref, b1_ref, w2_ref, b2_ref, o_ref):
    xb = x_ref[0]
    cx = jnp.dot(fwd_ref[...], xb,
                 preferred_element_type=jnp.float32, precision=jax.lax.Precision.HIGHEST)
    cr = cx[:_FP]
    ci = cx[_FP:]
    mag = jnp.sqrt(cr * cr + ci * ci)

    bits = jax.lax.bitcast_convert_type(mag, jnp.int32)
    ans = jnp.zeros((1, _C), jnp.int32)
    for b in range(30, -1, -1):
        t = ans | (1 << b)
        cnt = jnp.sum((bits < t).astype(jnp.int32), axis=0, keepdims=True)
        ans = jnp.where(cnt <= _K_LO, t, ans)
    s_lo = jax.lax.bitcast_convert_type(ans, jnp.float32)
    cnt_le = jnp.sum((bits <= ans).astype(jnp.int32), axis=0, keepdims=True)
    above = jnp.where(mag > s_lo, mag, jnp.inf)
    s_hi = jnp.where(cnt_le >= _K_LO + 2, s_lo,
                     jnp.min(above, axis=0, keepdims=True))
    q = s_lo * _LW + s_hi * _HW

    cond = mag < q
    crm = jnp.where(cond, mr_ref[...], cr)
    cim = jnp.where(cond, mi_ref[...], ci)

    dx = (jnp.dot(invc_ref[...], crm, preferred_element_type=jnp.float32, precision=jax.lax.Precision.HIGHEST) +
          jnp.dot(invs_ref[...], cim, preferred_element_type=jnp.float32, precision=jax.lax.Precision.HIGHEST))

    h = jnp.dot(dx, w1_ref[...], preferred_element_type=jnp.float32, precision=jax.lax.Precision.HIGHEST) + b1_ref[...]
    h = 0.5 * h * (1.0 + jax.lax.erf(h * np.float32(1.0 / np.sqrt(2.0))))
    r = jnp.dot(h, w2_ref[...], preferred_element_type=jnp.float32, precision=jax.lax.Precision.HIGHEST) + b2_ref[...]
    o_ref[0] = jax.nn.sigmoid(r)


def kernel(x, mask_real, mask_imag, W1, b1, W2, b2):
    B = x.shape[0]
    mr = mask_real.reshape(1, _C)
    mi = mask_imag.reshape(1, _C)
    b1r = b1.reshape(1, _C)
    b2r = b2.reshape(1, _C)
    const = lambda bs: pl.BlockSpec(bs, lambda b: (0,) * len(bs))
    return pl.pallas_call(
        _body,
        grid=(B,),
        in_specs=[
            pl.BlockSpec((1, _N, _C), lambda b: (b, 0, 0)),
            const((2 * _FP, _N)),
            const((_N, _FP)),
            const((_N, _FP)),
            const((1, _C)),
            const((1, _C)),
            const((_C, _C)),
            const((1, _C)),
            const((_C, _C)),
            const((1, _C)),
        ],
        out_specs=pl.BlockSpec((1, _N, _C), lambda b: (b, 0, 0)),
        out_shape=jax.ShapeDtypeStruct((B, _N, _C), jnp.float32),
        compiler_params=pltpu.CompilerParams(
            dimension_semantics=("arbitrary",),
            vmem_limit_bytes=120 * 1024 * 1024,
        ),
    )(x, jnp.asarray(_FWD_NP), jnp.asarray(_INVC_NP), jnp.asarray(_INVS_NP),
      mr, mi, W1, b1r, W2, b2r)


# speedup vs baseline: 2.2361x; 1.7456x over previous
import numpy as np
import jax
import jax.numpy as jnp
from jax.experimental import pallas as pl
from jax.experimental.pallas import tpu as pltpu

_N = 2048
_F = 1025
_FP = 1032
_C = 128
_BB = 2

_K_LO = 716 + (_FP - _F)

_IDX = np.float32(0.7) * np.float32(_F - 1)
_LW = np.float32(np.float32(717.0) - _IDX)
_HW = np.float32(_IDX - np.float32(716.0))

_HI = jax.lax.Precision.HIGHEST


def _build_dft():
    f = np.arange(_FP, dtype=np.int64)[:, None]
    w = np.arange(_N, dtype=np.int64)[None, :]
    ang = (2.0 * np.pi / _N) * ((f * w) % _N).astype(np.float64)
    cos = np.cos(ang)
    sin = np.sin(ang)
    cos[_F:, :] = 0.0
    sin[_F:, :] = 0.0
    fwd = np.concatenate([cos, -sin], axis=0).astype(np.float32)
    wgt = np.full((_FP, 1), 2.0 / _N)
    wgt[0] = 1.0 / _N
    wgt[_N // 2] = 1.0 / _N
    wgt[_F:] = 0.0
    inv_c = (wgt * cos).T.astype(np.float32)
    inv_s = (-(wgt * sin)).T.astype(np.float32)
    return fwd, inv_c, inv_s


_FWD_NP, _INVC_NP, _INVS_NP = _build_dft()


def _body(x_ref, fwd_ref, invc_ref, invs_ref, mr_ref, mi_ref,
          w1_ref, b1_ref, w2_ref, b2_ref, o_ref):
    xb = jnp.concatenate([x_ref[i] for i in range(_BB)], axis=1)
    cx = jnp.dot(fwd_ref[...], xb,
                 preferred_element_type=jnp.float32, precision=_HI)
    cr = cx[:_FP]
    ci = cx[_FP:]
    mag = jnp.sqrt(cr * cr + ci * ci)

    bits = jax.lax.bitcast_convert_type(mag, jnp.int32)
    ans = jnp.zeros((1, _BB * _C), jnp.int32)
    for b in range(30, -1, -1):
        t = ans | (1 << b)
        cnt = jnp.sum((bits < t).astype(jnp.int32), axis=0, keepdims=True)
        ans = jnp.where(cnt <= _K_LO, t, ans)
    s_lo = jax.lax.bitcast_convert_type(ans, jnp.float32)
    cnt_le = jnp.sum((bits <= ans).astype(jnp.int32), axis=0, keepdims=True)
    above = jnp.where(mag > s_lo, mag, jnp.inf)
    s_hi = jnp.where(cnt_le >= _K_LO + 2, s_lo,
                     jnp.min(above, axis=0, keepdims=True))
    q = s_lo * _LW + s_hi * _HW

    cond = mag < q
    crm = jnp.where(cond, mr_ref[...], cr)
    cim = jnp.where(cond, mi_ref[...], ci)

    dx = (jnp.dot(invc_ref[...], crm,
                  preferred_element_type=jnp.float32, precision=_HI) +
          jnp.dot(invs_ref[...], cim,
                  preferred_element_type=jnp.float32, precision=_HI))

    for i in range(_BB):
        dxi = dx[:, i * _C:(i + 1) * _C]
        h = jnp.dot(dxi, w1_ref[...],
                    preferred_element_type=jnp.float32, precision=_HI)
        h = h + b1_ref[...]
        h = 0.5 * h * (1.0 + jax.lax.erf(h * np.float32(1.0 / np.sqrt(2.0))))
        r = jnp.dot(h, w2_ref[...],
                    preferred_element_type=jnp.float32, precision=_HI)
        o_ref[i] = jax.nn.sigmoid(r + b2_ref[...])


def kernel(x, mask_real, mask_imag, W1, b1, W2, b2):
    B = x.shape[0]
    mr = jnp.tile(mask_real.reshape(1, _C), (1, _BB))
    mi = jnp.tile(mask_imag.reshape(1, _C), (1, _BB))
    b1r = b1.reshape(1, _C)
    b2r = b2.reshape(1, _C)
    const = lambda bs: pl.BlockSpec(bs, lambda b: (0,) * len(bs))
    return pl.pallas_call(
        _body,
        grid=(B // _BB,),
        in_specs=[
            pl.BlockSpec((_BB, _N, _C), lambda b: (b, 0, 0)),
            const((2 * _FP, _N)),
            const((_N, _FP)),
            const((_N, _FP)),
            const((1, _BB * _C)),
            const((1, _BB * _C)),
            const((_C, _C)),
            const((1, _C)),
            const((_C, _C)),
            const((1, _C)),
        ],
        out_specs=pl.BlockSpec((_BB, _N, _C), lambda b: (b, 0, 0)),
        out_shape=jax.ShapeDtypeStruct((B, _N, _C), jnp.float32),
        compiler_params=pltpu.CompilerParams(
            dimension_semantics=("arbitrary",),
            vmem_limit_bytes=120 * 1024 * 1024,
        ),
    )(x, jnp.asarray(_FWD_NP), jnp.asarray(_INVC_NP), jnp.asarray(_INVS_NP),
      mr, mi, W1, b1r, W2, b2r)


# speedup vs baseline: 5.5326x; 2.4743x over previous
import numpy as np
import jax
import jax.numpy as jnp
from jax.experimental import pallas as pl
from jax.experimental.pallas import tpu as pltpu

_N = 2048
_H = 1024
_KP = 520
_C = 128
_BB = 2
_NC = _BB * _C

_K_LO = 716

_IDX = np.float32(0.7) * np.float32(1024.0)
_LW = np.float32(np.float32(717.0) - _IDX)
_HW = np.float32(_IDX - np.float32(716.0))

_HI = jax.lax.Precision.HIGHEST


def _build_mats():
    k = np.arange(_KP, dtype=np.int64)[:, None]
    n = np.arange(_H, dtype=np.int64)[None, :]
    ang_e = (2.0 * np.pi / _N) * ((k * (2 * n)) % _N).astype(np.float64)
    ang_o = (2.0 * np.pi / _N) * ((k * (2 * n + 1)) % _N).astype(np.float64)
    valid = (k <= 512).astype(np.float64)
    me = np.concatenate([np.cos(ang_e) * valid, -np.sin(ang_e) * valid], 0)
    mo = np.concatenate([np.cos(ang_o) * valid, -np.sin(ang_o) * valid], 0)
    wgt = np.full((_KP, 1), 2.0 / _N)
    wgt[0] = 1.0 / _N
    wgt[512] = 1.0 / _N
    wgt[513:] = 0.0
    jc = (wgt * np.cos(ang_e)).T
    js = (-(wgt * np.sin(ang_e))).T
    jco = (wgt * np.cos(ang_o)).T
    jso = (-(wgt * np.sin(ang_o))).T
    f32 = np.float32
    return (me.astype(f32), mo.astype(f32), jc.astype(f32), js.astype(f32),
            jco.astype(f32), jso.astype(f32))


_ME, _MO, _JC, _JS, _JCO, _JSO = _build_mats()


def _body(x_ref, me_ref, mo_ref, jc_ref, js_ref, jco_ref, jso_ref,
          mr_ref, mi_ref, w1_ref, b1_ref, w2_ref, b2_ref, o_ref):
    xe = jnp.concatenate([x_ref[i, :, :_C] for i in range(_BB)], axis=1)
    xo = jnp.concatenate([x_ref[i, :, _C:] for i in range(_BB)], axis=1)
    e = jnp.dot(me_ref[...], xe, preferred_element_type=jnp.float32,
                precision=_HI)
    op = jnp.dot(mo_ref[...], xo, preferred_element_type=jnp.float32,
                 precision=_HI)
    xlo = e + op
    d = e - op
    xlr, xli = xlo[:_KP], xlo[_KP:]
    dr, di = d[:_KP], d[_KP:]

    riota = jax.lax.broadcasted_iota(jnp.int32, (_KP, _NC), 0)
    inf = jnp.float32(jnp.inf)
    mag_lo = jnp.sqrt(xlr * xlr + xli * xli)
    mag_d = jnp.sqrt(dr * dr + di * di)
    mag_lo = jnp.where(riota > 512, inf, mag_lo)
    mag_d = jnp.where(riota > 511, inf, mag_d)
    smag = jnp.concatenate([mag_lo, mag_d], axis=0)

    bits = jax.lax.bitcast_convert_type(smag, jnp.int32)
    ans = jnp.zeros((1, _NC), jnp.int32)
    for b in range(30, -1, -1):
        t = ans | (1 << b)
        cnt = jnp.sum((bits < t).astype(jnp.int32), axis=0, keepdims=True)
        ans = jnp.where(cnt <= _K_LO, t, ans)
    s_lo = jax.lax.bitcast_convert_type(ans, jnp.float32)
    cnt_le = jnp.sum((bits <= ans).astype(jnp.int32), axis=0, keepdims=True)
    above = jnp.where(smag > s_lo, smag, inf)
    s_hi = jnp.where(cnt_le >= _K_LO + 2, s_lo,
                     jnp.min(above, axis=0, keepdims=True))
    q = s_lo * _LW + s_hi * _HW

    tr = mr_ref[...]
    ti = mi_ref[...]
    cond_lo = mag_lo < q
    cond_d = mag_d < q
    clr = jnp.where(cond_lo, tr, xlr)
    cli = jnp.where(cond_lo, ti, xli)
    cdr = jnp.where(cond_d, tr, dr)
    cdi = jnp.where(cond_d, -ti, di)
    ar = clr + cdr
    ai = cli + cdi
    gr = clr - cdr
    gi = cli - cdi
    is512 = riota == 512
    zero = jnp.float32(0.0)
    ar = jnp.where(is512, 2.0 * clr, ar)
    ai = jnp.where(is512, zero, ai)
    gr = jnp.where(is512, zero, gr)
    gi = jnp.where(is512, 2.0 * cli, gi)

    dxe = (jnp.dot(jc_ref[...], ar, preferred_element_type=jnp.float32,
                   precision=_HI) +
           jnp.dot(js_ref[...], ai, preferred_element_type=jnp.float32,
                   precision=_HI))
    dxo = (jnp.dot(jco_ref[...], gr, preferred_element_type=jnp.float32,
                   precision=_HI) +
           jnp.dot(jso_ref[...], gi, preferred_element_type=jnp.float32,
                   precision=_HI))

    for i in range(_BB):
        sl = slice(i * _C, (i + 1) * _C)
        dcat = jnp.concatenate([dxe[:, sl], dxo[:, sl]], axis=0)
        h = jnp.dot(dcat, w1_ref[...], preferred_element_type=jnp.float32,
                    precision=_HI) + b1_ref[...]
        h = 0.5 * h * (1.0 + jax.lax.erf(h * np.float32(1.0 / np.sqrt(2.0))))
        r = jnp.dot(h, w2_ref[...], preferred_element_type=jnp.float32,
                    precision=_HI) + b2_ref[...]
        rec = jax.nn.sigmoid(r)
        o_ref[i] = jnp.concatenate([rec[:_H], rec[_H:]], axis=1)


def kernel(x, mask_real, mask_imag, W1, b1, W2, b2):
    B = x.shape[0]
    xr = x.reshape(B, _H, 2 * _C)
    mr = jnp.tile(mask_real.reshape(1, _C), (1, _BB))
    mi = jnp.tile(mask_imag.reshape(1, _C), (1, _BB))
    b1r = b1.reshape(1, _C)
    b2r = b2.reshape(1, _C)
    const = lambda bs: pl.BlockSpec(bs, lambda b: (0,) * len(bs))
    out = pl.pallas_call(
        _body,
        grid=(B // _BB,),
        in_specs=[
            pl.BlockSpec((_BB, _H, 2 * _C), lambda b: (b, 0, 0)),
            const((2 * _KP, _H)),
            const((2 * _KP, _H)),
            const((_H, _KP)),
            const((_H, _KP)),
            const((_H, _KP)),
            const((_H, _KP)),
            const((1, _NC)),
            const((1, _NC)),
            const((_C, _C)),
            const((1, _C)),
            const((_C, _C)),
            const((1, _C)),
        ],
        out_specs=pl.BlockSpec((_BB, _H, 2 * _C), lambda b: (b, 0, 0)),
        out_shape=jax.ShapeDtypeStruct((B, _H, 2 * _C), jnp.float32),
        compiler_params=pltpu.CompilerParams(
            dimension_semantics=("arbitrary",),
            vmem_limit_bytes=120 * 1024 * 1024,
        ),
    )(xr, jnp.asarray(_ME), jnp.asarray(_MO), jnp.asarray(_JC),
      jnp.asarray(_JS), jnp.asarray(_JCO), jnp.asarray(_JSO),
      mr, mi, W1, b1r, W2, b2r)
    return out.reshape(B, _N, _C)
